# Initial kernel scaffold; baseline (speedup 1.0000x reference)
import jax
import jax.numpy as jnp
from jax.experimental import pallas as pl


def kernel(input, W_real, W_imag):
    raise NotImplementedError("write your pallas kernel here")


# speedup vs baseline: 1.0057x; 1.0057x over previous
import functools

import jax
import jax.numpy as jnp
from jax.experimental import pallas as pl
from jax.experimental.pallas import tpu as pltpu
from jax.experimental.pallas import tpu_sc as plsc

_WINDOW = 128


@functools.partial(jax.jit, static_argnums=())
def _sc_gather2(W_real, W_imag, idx2d):
    B = idx2d.shape[1]
    D = W_real.shape[1]
    mesh = plsc.VectorSubcoreMesh(core_axis_name="c", subcore_axis_name="s")

    @functools.partial(
        pl.kernel,
        out_type=[
            jax.ShapeDtypeStruct((B, D), jnp.float32),
            jax.ShapeDtypeStruct((B, D), jnp.float32),
        ],
        mesh=mesh,
        compiler_params=pltpu.CompilerParams(use_tc_tiling_on_sc=False),
    )
    def k(wr_hbm, wi_hbm, idx_hbm, r_hbm, i_hbm):
        def body(idx_v, r_v, i_v):
            pltpu.sync_copy(wr_hbm.at[idx_v.at[0]], r_v)
            pltpu.sync_copy(wi_hbm.at[idx_v.at[0]], i_v)

        pltpu.emit_pipeline(
            body,
            grid=(B // _WINDOW,),
            in_specs=[
                pl.BlockSpec((1, _WINDOW), index_map=lambda w: (0, w)),
            ],
            out_specs=[
                pl.BlockSpec((_WINDOW, D), index_map=lambda w: (w, 0)),
                pl.BlockSpec((_WINDOW, D), index_map=lambda w: (w, 0)),
            ],
            core_axis_name=("c", "s"),
            dimension_semantics=(pltpu.PARALLEL,),
        )(idx_hbm, r_hbm, i_hbm)

    return k(W_real, W_imag, idx2d)


def kernel(input, W_real, W_imag):
    BATCH, HIST = input.shape
    D = W_real.shape[1]
    idx2d = input.reshape(1, BATCH * HIST)
    r, i = _sc_gather2(W_real, W_imag, idx2d)
    out = jax.lax.complex(r, i)
    return out.reshape(BATCH, HIST, D)


# speedup vs baseline: 1.0854x; 1.0792x over previous
import functools

import jax
import jax.numpy as jnp
from jax.experimental import pallas as pl
from jax.experimental.pallas import tpu as pltpu
from jax.experimental.pallas import tpu_sc as plsc

_WINDOW = 512


@functools.partial(jax.jit, static_argnums=())
def _sc_gather2(W_real, W_imag, idx2d):
    B = idx2d.shape[1]
    D = W_real.shape[1]
    mesh = plsc.VectorSubcoreMesh(core_axis_name="c", subcore_axis_name="s")

    NW = B // _WINDOW

    @functools.partial(
        pl.kernel,
        out_type=[
            jax.ShapeDtypeStruct((NW, _WINDOW, D), jnp.float32),
            jax.ShapeDtypeStruct((NW, _WINDOW, D), jnp.float32),
        ],
        mesh=mesh,
        compiler_params=pltpu.CompilerParams(use_tc_tiling_on_sc=False),
    )
    def k(wr_hbm, wi_hbm, idx_hbm, r_hbm, i_hbm):
        def body(idx_v, r_v, i_v):
            pltpu.sync_copy(wr_hbm.at[idx_v.at[0]], r_v.at[0])
            pltpu.sync_copy(wi_hbm.at[idx_v.at[0]], i_v.at[0])

        pltpu.emit_pipeline(
            body,
            grid=(NW,),
            in_specs=[
                pl.BlockSpec((1, _WINDOW), index_map=lambda w: (0, w)),
            ],
            out_specs=[
                pl.BlockSpec((1, _WINDOW, D), index_map=lambda w: (w, 0, 0)),
                pl.BlockSpec((1, _WINDOW, D), index_map=lambda w: (w, 0, 0)),
            ],
            core_axis_name=("c", "s"),
            dimension_semantics=(pltpu.PARALLEL,),
        )(idx_hbm, r_hbm, i_hbm)

    return k(W_real, W_imag, idx2d)


def kernel(input, W_real, W_imag):
    BATCH, HIST = input.shape
    D = W_real.shape[1]
    idx2d = input.reshape(1, BATCH * HIST)
    r, i = _sc_gather2(W_real, W_imag, idx2d)
    out = jax.lax.complex(r.reshape(-1), i.reshape(-1))
    return out.reshape(BATCH, HIST, D)
